# Initial kernel scaffold; baseline (speedup 1.0000x reference)
import jax
import jax.numpy as jnp
from jax.experimental import pallas as pl


def kernel(x, map_table):
    raise NotImplementedError("write your pallas kernel here")


# speedup vs baseline: 5.1952x; 5.1952x over previous
import dataclasses
import functools

import jax
import jax.numpy as jnp
from jax import lax
from jax.experimental import pallas as pl
from jax.experimental.pallas import tpu as pltpu
from jax.experimental.pallas import tpu_sc as plsc

_B, _S = 16384, 200
_NTOK = _B * _S
_OUT_WORDS = _NTOK * 4
_CHUNK = 6400
_GRID = _NTOK // _CHUNK
_LANES = 16


def _sc_lookup(x_flat, tbl_pad):
    mesh = plsc.VectorSubcoreMesh(core_axis_name="c", subcore_axis_name="s")
    cp = pltpu.CompilerParams()
    if "needs_layout_passes" in pltpu.CompilerParams.__dataclass_fields__:
        cp = dataclasses.replace(cp, needs_layout_passes=False)

    @functools.partial(
        pl.kernel,
        out_type=jax.ShapeDtypeStruct((_OUT_WORDS,), jnp.float32),
        mesh=mesh,
        scratch_types=[pltpu.VMEM((64,), jnp.float32)],
        compiler_params=cp,
    )
    def k(x_hbm, tbl_hbm, out_hbm, tbl_v):
        pltpu.sync_copy(tbl_hbm, tbl_v)

        def body(x_vmem, o_vmem):
            lane = lax.broadcasted_iota(jnp.int32, (_LANES,), 0)
            opos0 = lane * 4

            @pl.loop(0, _CHUNK // _LANES)
            def _(j):
                xv = x_vmem[pl.ds(j * _LANES, _LANES)]
                base = xv * 4
                op = j * (4 * _LANES) + opos0
                for c in range(4):
                    vals = plsc.load_gather(tbl_v, [base + c])
                    plsc.store_scatter(o_vmem, [op + c], vals)

        pltpu.emit_pipeline(
            body,
            grid=(_GRID,),
            in_specs=[pl.BlockSpec((_CHUNK,), lambda i: (i,))],
            out_specs=[pl.BlockSpec((_CHUNK * 4,), lambda i: (i,))],
            core_axis_name=("c", "s"),
            dimension_semantics=(pltpu.PARALLEL,),
        )(x_hbm, out_hbm)

    return k(x_flat, tbl_pad)


@jax.jit
def kernel(x, map_table):
    x_flat = x.reshape(_NTOK)
    tbl_pad = jnp.zeros((64,), jnp.float32).at[:36].set(map_table.reshape(36))
    out = _sc_lookup(x_flat, tbl_pad)
    return out.reshape(_B, _S, 4)


# speedup vs baseline: 69.8867x; 13.4522x over previous
import dataclasses
import functools

import jax
import jax.numpy as jnp
from jax.experimental import pallas as pl
from jax.experimental.pallas import tpu as pltpu
from jax.experimental.pallas import tpu_sc as plsc

_B, _S = 16384, 200
_LANES = 16
_RB = 8
_DC = 512
_GRID_R = _S // _RB
_GRID_C = _B // _DC


def _sc_lookup(xt, tbl_pad):
    mesh = plsc.VectorSubcoreMesh(core_axis_name="c", subcore_axis_name="s")
    cp = pltpu.CompilerParams()
    if "needs_layout_passes" in pltpu.CompilerParams.__dataclass_fields__:
        cp = dataclasses.replace(cp, needs_layout_passes=False)

    @functools.partial(
        pl.kernel,
        out_type=jax.ShapeDtypeStruct((_S, _B // 32, 128), jnp.float32),
        mesh=mesh,
        scratch_types=[pltpu.VMEM((64,), jnp.float32)],
        compiler_params=cp,
    )
    def k(x_hbm, tbl_hbm, out_hbm, tbl_v):
        pltpu.sync_copy(tbl_hbm, tbl_v)

        def body(x_vmem, o_vmem):
            @pl.loop(0, _RB)
            def _(r):
                @pl.loop(0, _DC // 128)
                def _(h):
                    @pl.loop(0, 128 // _LANES)
                    def _(i):
                        xv = x_vmem[r, pl.ds(h * 128 + i * _LANES, _LANES)]
                        base = xv * 4
                        for c in range(4):
                            vals = plsc.load_gather(tbl_v, [base + c])
                            o_vmem[r, h * 4 + c, pl.ds(i * _LANES, _LANES)] = vals

        pltpu.emit_pipeline(
            body,
            grid=(_GRID_R, _GRID_C),
            in_specs=[pl.BlockSpec((_RB, _DC), lambda i, j: (i, j))],
            out_specs=[
                pl.BlockSpec((_RB, _DC // 128 * 4, 128), lambda i, j: (i, j, 0))
            ],
            core_axis_name=("c", "s"),
            dimension_semantics=(pltpu.PARALLEL, pltpu.PARALLEL),
        )(x_hbm, out_hbm)

    return k(xt, tbl_pad)


@jax.jit
def kernel(x, map_table):
    tbl_pad = jnp.zeros((64,), jnp.float32).at[:36].set(map_table.reshape(36))
    p = _sc_lookup(x.T, tbl_pad)
    q = p.reshape(_S, _B // 128, 4, 128)
    return q.transpose(1, 3, 0, 2).reshape(_B, _S, 4)


# speedup vs baseline: 69.9118x; 1.0004x over previous
import dataclasses
import functools

import jax
import jax.numpy as jnp
from jax.experimental import pallas as pl
from jax.experimental.pallas import tpu as pltpu
from jax.experimental.pallas import tpu_sc as plsc

_B, _S = 16384, 200
_LANES = 16
_RB = 8
_DC = 512
_GRID_R = _S // _RB
_GRID_C = _B // _DC


def _sc_lookup(xt, tbl_pad):
    mesh = plsc.VectorSubcoreMesh(core_axis_name="c", subcore_axis_name="s")
    cp = pltpu.CompilerParams()
    if "needs_layout_passes" in pltpu.CompilerParams.__dataclass_fields__:
        cp = dataclasses.replace(cp, needs_layout_passes=False)

    @functools.partial(
        pl.kernel,
        out_type=jax.ShapeDtypeStruct((_S, _B // 32, 128), jnp.float32),
        mesh=mesh,
        scratch_types=[pltpu.VMEM((64,), jnp.float32)],
        compiler_params=cp,
    )
    def k(x_hbm, tbl_hbm, out_hbm, tbl_v):
        pltpu.sync_copy(tbl_hbm, tbl_v)

        def body(x_vmem, o_vmem):
            @pl.loop(0, _RB)
            def _(r):
                @pl.loop(0, _DC // 128)
                def _(h):
                    for i in range(128 // _LANES):
                        xv = x_vmem[r, pl.ds(h * 128 + i * _LANES, _LANES)]
                        base = xv * 4
                        for c in range(4):
                            vals = plsc.load_gather(tbl_v, [base + c])
                            o_vmem[r, h * 4 + c, pl.ds(i * _LANES, _LANES)] = vals

        pltpu.emit_pipeline(
            body,
            grid=(_GRID_R, _GRID_C),
            in_specs=[pl.BlockSpec((_RB, _DC), lambda i, j: (i, j))],
            out_specs=[
                pl.BlockSpec((_RB, _DC // 128 * 4, 128), lambda i, j: (i, j, 0))
            ],
            core_axis_name=("c", "s"),
            dimension_semantics=(pltpu.PARALLEL, pltpu.PARALLEL),
        )(x_hbm, out_hbm)

    return k(xt, tbl_pad)


@jax.jit
def kernel(x, map_table):
    tbl_pad = jnp.zeros((64,), jnp.float32).at[:36].set(map_table.reshape(36))
    p = _sc_lookup(x.T, tbl_pad)
    q = p.reshape(_S, _B // 128, 4, 128)
    return q.transpose(1, 3, 0, 2).reshape(_B, _S, 4)


# speedup vs baseline: 83.7535x; 1.1980x over previous
import dataclasses
import functools

import jax
import jax.numpy as jnp
from jax.experimental import pallas as pl
from jax.experimental.pallas import tpu as pltpu
from jax.experimental.pallas import tpu_sc as plsc

_B, _S = 16384, 200
_LANES = 16
_RB = 8
_DC = 512
_GRID_R = _S // _RB
_GRID_C = _B // _DC


def _sc_lookup(xt, tbl_pad):
    mesh = plsc.VectorSubcoreMesh(core_axis_name="c", subcore_axis_name="s")
    cp = pltpu.CompilerParams()
    if "needs_layout_passes" in pltpu.CompilerParams.__dataclass_fields__:
        cp = dataclasses.replace(cp, needs_layout_passes=False)

    @functools.partial(
        pl.kernel,
        out_type=jax.ShapeDtypeStruct((_S, _B // 32, 128), jnp.float32),
        mesh=mesh,
        scratch_types=[
            pltpu.VMEM((64,), jnp.float32),
            pltpu.VMEM((576,), jnp.float32),
        ],
        compiler_params=cp,
    )
    def k(x_hbm, tbl_hbm, out_hbm, tbl_v, ts_v):
        pltpu.sync_copy(tbl_hbm, tbl_v)
        for t in range(36):
            val = plsc.load_gather(tbl_v, [jnp.full((_LANES,), t, jnp.int32)])
            ts_v[pl.ds(t * _LANES, _LANES)] = val

        def body(x_vmem, o_vmem):
            lane = jax.lax.broadcasted_iota(jnp.int32, (_LANES,), 0)
            lane_c = [lane + _LANES * c for c in range(4)]

            @pl.loop(0, _RB)
            def _(r):
                @pl.loop(0, _DC // 128)
                def _(h):
                    for i in range(128 // _LANES):
                        xv = x_vmem[r, pl.ds(h * 128 + i * _LANES, _LANES)]
                        base = xv * 64
                        for c in range(4):
                            vals = plsc.load_gather(ts_v, [base + lane_c[c]])
                            o_vmem[r, h * 4 + c, pl.ds(i * _LANES, _LANES)] = vals

        pltpu.emit_pipeline(
            body,
            grid=(_GRID_R, _GRID_C),
            in_specs=[pl.BlockSpec((_RB, _DC), lambda i, j: (i, j))],
            out_specs=[
                pl.BlockSpec((_RB, _DC // 128 * 4, 128), lambda i, j: (i, j, 0))
            ],
            core_axis_name=("c", "s"),
            dimension_semantics=(pltpu.PARALLEL, pltpu.PARALLEL),
        )(x_hbm, out_hbm)

    return k(xt, tbl_pad)


@jax.jit
def kernel(x, map_table):
    tbl_pad = jnp.zeros((64,), jnp.float32).at[:36].set(map_table.reshape(36))
    p = _sc_lookup(x.T, tbl_pad)
    q = p.reshape(_S, _B // 128, 4, 128)
    return q.transpose(1, 3, 0, 2).reshape(_B, _S, 4)
